# Initial kernel scaffold; baseline (speedup 1.0000x reference)
import jax
import jax.numpy as jnp
from jax.experimental import pallas as pl


def kernel(x, edge_attr, feat_mask, equality, edge_index, batch, W_atom, b_atom, W_rbf, b_rbf, Wq1, Wk1, Wv1, Wq2, Wk2, Wv2, Wm, We1, We2, Ew1, Eb1, Ew2, Eb2):
    raise NotImplementedError("write your pallas kernel here")


# speedup vs baseline: 1.3921x; 1.3921x over previous
import dataclasses
import functools
import math

import jax
import jax.numpy as jnp
from jax import lax
from jax.experimental import pallas as pl
from jax.experimental.pallas import tpu as pltpu
from jax.experimental.pallas import tpu_sc as plsc

EMB = 128
BINS = 512
BCR = 16
L = 36
N = 10000
E = 320000
NSC = 2
NTL = 16
NWK = NSC * NTL
EPW = E // NWK
GCH = 400
GCH2 = 200

_MESH = plsc.VectorSubcoreMesh(core_axis_name="c", subcore_axis_name="s")

_SC_PARAMS = pltpu.CompilerParams()
if "needs_layout_passes" in pltpu.CompilerParams.__dataclass_fields__:
    _SC_PARAMS = dataclasses.replace(_SC_PARAMS, needs_layout_passes=False)

_f32 = jnp.float32


def _dot(a, b):
    return jnp.dot(a, b, preferred_element_type=_f32)


def _e_body(gam_ref, dd_ref, cen_ref, wr_ref, br_ref, e_ref):
    dd = dd_ref[...]
    diff = dd - cen_ref[...]
    g = gam_ref[0, 0]
    rbf = jnp.exp(-g * (diff * diff))
    z = _dot(rbf, wr_ref[...]) + br_ref[...]
    e_ref[...] = jax.nn.softplus(z)


def _e_tc(gam, dd, cen, W_rbf, b_rbf):
    EB = 512
    return pl.pallas_call(
        _e_body,
        grid=(E // EB,),
        in_specs=[
            pl.BlockSpec(memory_space=pltpu.SMEM),
            pl.BlockSpec((EB, 1), lambda i: (i, 0)),
            pl.BlockSpec((1, BINS), lambda i: (0, 0)),
            pl.BlockSpec((BINS, EMB), lambda i: (0, 0)),
            pl.BlockSpec((1, EMB), lambda i: (0, 0)),
        ],
        out_specs=pl.BlockSpec((EB, EMB), lambda i: (i, 0)),
        out_shape=jax.ShapeDtypeStruct((E, EMB), _f32),
    )(gam, dd, cen, W_rbf, b_rbf)


def _prep_body(x_ref, wa_ref, ba_ref, wq_ref, wk_ref, wv_ref,
               node_ref, q_ref, k_ref, v_ref):
    n0 = _dot(x_ref[...], wa_ref[...]) + ba_ref[...]
    node_ref[...] = n0
    q_ref[...] = _dot(n0, wq_ref[...])
    k_ref[...] = _dot(n0, wk_ref[...])
    v_ref[...] = _dot(n0, wv_ref[...])


def _prep_tc(x, W_atom, b_atom, Wq, Wk, Wv):
    BLK = 400
    outs = [jax.ShapeDtypeStruct((N, EMB), _f32)] * 4
    return pl.pallas_call(
        _prep_body,
        grid=(N // BLK,),
        in_specs=[
            pl.BlockSpec((BLK, 92), lambda i: (i, 0)),
            pl.BlockSpec((92, EMB), lambda i: (0, 0)),
            pl.BlockSpec((1, EMB), lambda i: (0, 0)),
            pl.BlockSpec((EMB, EMB), lambda i: (0, 0)),
            pl.BlockSpec((EMB, EMB), lambda i: (0, 0)),
            pl.BlockSpec((EMB, EMB), lambda i: (0, 0)),
        ],
        out_specs=[pl.BlockSpec((BLK, EMB), lambda i: (i, 0))] * 4,
        out_shape=outs,
    )(x, W_atom, b_atom, Wq, Wk, Wv)


def _inter_body(node_ref, agg_ref, wq_ref, wk_ref, wv_ref,
                node1_ref, q_ref, k_ref, v_ref):
    n1 = node_ref[...] + agg_ref[...]
    node1_ref[...] = n1
    q_ref[...] = _dot(n1, wq_ref[...])
    k_ref[...] = _dot(n1, wk_ref[...])
    v_ref[...] = _dot(n1, wv_ref[...])


def _inter_tc(node, agg, Wq, Wk, Wv):
    BLK = 400
    outs = [jax.ShapeDtypeStruct((N, EMB), _f32)] * 4
    return pl.pallas_call(
        _inter_body,
        grid=(N // BLK,),
        in_specs=[
            pl.BlockSpec((BLK, EMB), lambda i: (i, 0)),
            pl.BlockSpec((BLK, EMB), lambda i: (i, 0)),
            pl.BlockSpec((EMB, EMB), lambda i: (0, 0)),
            pl.BlockSpec((EMB, EMB), lambda i: (0, 0)),
            pl.BlockSpec((EMB, EMB), lambda i: (0, 0)),
        ],
        out_specs=[pl.BlockSpec((BLK, EMB), lambda i: (i, 0))] * 4,
        out_shape=outs,
    )(node, agg, Wq, Wk, Wv)


def _inter2_body(node_ref, agg_ref, wm_ref, we1_ref, m_ref, t1_ref):
    n2 = node_ref[...] + agg_ref[...]
    m_ref[...] = _dot(n2, wm_ref[...])
    t1_ref[...] = _dot(n2, we1_ref[...])


def _inter2_tc(node, agg, Wm, We1):
    BLK = 400
    outs = [jax.ShapeDtypeStruct((N, EMB), _f32)] * 2
    return pl.pallas_call(
        _inter2_body,
        grid=(N // BLK,),
        in_specs=[
            pl.BlockSpec((BLK, EMB), lambda i: (i, 0)),
            pl.BlockSpec((BLK, EMB), lambda i: (i, 0)),
            pl.BlockSpec((EMB, EMB), lambda i: (0, 0)),
            pl.BlockSpec((EMB, EMB), lambda i: (0, 0)),
        ],
        out_specs=[pl.BlockSpec((BLK, EMB), lambda i: (i, 0))] * 2,
        out_shape=outs,
    )(node, agg, Wm, We1)


def _node3_body(t1_ref, agg_ref, we2_ref, o_ref):
    o_ref[...] = jax.nn.softplus(t1_ref[...] + _dot(agg_ref[...], we2_ref[...]))


def _node3_tc(t1, agg3, We2):
    BLK = 400
    return pl.pallas_call(
        _node3_body,
        grid=(N // BLK,),
        in_specs=[
            pl.BlockSpec((BLK, EMB), lambda i: (i, 0)),
            pl.BlockSpec((BLK, EMB), lambda i: (i, 0)),
            pl.BlockSpec((EMB, EMB), lambda i: (0, 0)),
        ],
        out_specs=pl.BlockSpec((BLK, EMB), lambda i: (i, 0)),
        out_shape=jax.ShapeDtypeStruct((N, EMB), _f32),
    )(t1, agg3, We2)


def _worker_id():
    cid = lax.axis_index("c")
    sid = lax.axis_index("s")
    return cid, sid, sid * NSC + cid


def _gather_body(t_hbm, i_hbm, o_hbm, idxb, rows):
    cid, sid, wid = _worker_id()
    base = wid * EPW

    @pl.loop(0, EPW // GCH)
    def _(ci):
        off = pl.multiple_of(base + ci * GCH, 8)
        pltpu.sync_copy(i_hbm.at[pl.ds(off, GCH)], idxb)
        pltpu.sync_copy(t_hbm.at[idxb], rows)
        pltpu.sync_copy(rows, o_hbm.at[pl.ds(off, GCH), :])


def _sc_gather(table, idx):
    fn = pl.kernel(
        _gather_body,
        out_type=jax.ShapeDtypeStruct((E, EMB), _f32),
        mesh=_MESH,
        scratch_types=[
            pltpu.VMEM((GCH,), jnp.int32),
            pltpu.VMEM((GCH, EMB), _f32),
        ],
        compiler_params=_SC_PARAMS,
    )
    return fn(table, idx)


def _gather2_body(ta_hbm, tb_hbm, i_hbm, oa_hbm, ob_hbm, idxb, rowsa, rowsb):
    cid, sid, wid = _worker_id()
    base = wid * EPW

    @pl.loop(0, EPW // GCH2)
    def _(ci):
        off = pl.multiple_of(base + ci * GCH2, 8)
        pltpu.sync_copy(i_hbm.at[pl.ds(off, GCH2)], idxb)
        pltpu.sync_copy(ta_hbm.at[idxb], rowsa)
        pltpu.sync_copy(tb_hbm.at[idxb], rowsb)
        pltpu.sync_copy(rowsa, oa_hbm.at[pl.ds(off, GCH2), :])
        pltpu.sync_copy(rowsb, ob_hbm.at[pl.ds(off, GCH2), :])


def _sc_gather2(ta, tb, idx):
    fn = pl.kernel(
        _gather2_body,
        out_type=[jax.ShapeDtypeStruct((E, EMB), _f32)] * 2,
        mesh=_MESH,
        scratch_types=[
            pltpu.VMEM((GCH2,), jnp.int32),
            pltpu.VMEM((GCH2, EMB), _f32),
            pltpu.VMEM((GCH2, EMB), _f32),
        ],
        compiler_params=_SC_PARAMS,
    )
    return fn(ta, tb, idx)


def _pairs_body(row_hbm, eq1_hbm, res_hbm, rbuf, eq1b):
    cid, sid, wid = _worker_id()

    @pl.when(jnp.logical_and(cid == 0, sid == 0))
    def _():
        pltpu.sync_copy(row_hbm, rbuf)
        pltpu.sync_copy(eq1_hbm, eq1b)
        r = [rbuf[pl.ds(j * BCR, BCR)] for j in range(L)]
        for j in range(L):
            for k in range(j + 1, L):
                rj = r[j]
                rk = r[k]
                absv = jnp.abs(rj - rk) * 0.5
                sj = jnp.where(rj < 0, -absv, absv)
                c = eq1b[pl.ds((j * L + k) * BCR, BCR)] > 0.5
                r[j] = jnp.where(c, sj, rj)
                r[k] = jnp.where(c, -sj, rk)
        for j in range(L):
            rbuf[pl.ds(j * BCR, BCR)] = r[j]
        pltpu.sync_copy(rbuf, res_hbm)


def _sc_pairs(rowT, eq1T):
    fn = pl.kernel(
        _pairs_body,
        out_type=jax.ShapeDtypeStruct((L * BCR,), _f32),
        mesh=_MESH,
        scratch_types=[
            pltpu.VMEM((L * BCR,), _f32),
            pltpu.VMEM((L * L * BCR,), _f32),
        ],
        compiler_params=_SC_PARAMS,
    )
    return fn(rowT, eq1T)


def kernel(x, edge_attr, feat_mask, equality, edge_index, batch,
           W_atom, b_atom, W_rbf, b_rbf, Wq1, Wk1, Wv1, Wq2, Wk2, Wv2,
           Wm, We1, We2, Ew1, Eb1, Ew2, Eb2):
    src = edge_index[0]
    dst = edge_index[1]
    inv = 1.0 / jnp.sqrt(float(EMB))

    dd = (-0.75 / jnp.linalg.norm(edge_attr, axis=1)).reshape(E, 1)
    centers = jnp.linspace(-4.0, 0.0, BINS)
    gam = (1.0 / (centers[1] - centers[0]) ** 2).reshape(1, 1)
    e = _e_tc(gam, dd, centers.reshape(1, BINS), W_rbf, b_rbf.reshape(1, EMB))

    node0, q1, k1, v1 = _prep_tc(x, W_atom, b_atom.reshape(1, EMB),
                                 Wq1, Wk1, Wv1)

    def conv(q, k, v, node):
        ks, vs = _sc_gather2(k, v, src)
        qd = _sc_gather(q, dst)
        logits = jnp.sum(qd * (ks + e), axis=-1) / jnp.sqrt(float(EMB))
        m = jax.ops.segment_max(logits, dst, num_segments=N)
        m = jnp.where(jnp.isfinite(m), m, 0.0)
        ex = jnp.exp(logits - m[dst])
        s = jax.ops.segment_sum(ex, dst, num_segments=N)
        alpha = ex / (s[dst] + 1e-16)
        msg = alpha[:, None] * (vs + e)
        return jax.ops.segment_sum(msg, dst, num_segments=N)

    agg1 = conv(q1, k1, v1, node0)
    node1, q2, k2, v2 = _inter_tc(node0, agg1, Wq2, Wk2, Wv2)
    agg2 = conv(q2, k2, v2, node1)
    mWm, t1 = _inter2_tc(node1, agg2, Wm, We1)

    ms = _sc_gather(mWm, src)
    msg = ms * e
    agg = jax.ops.segment_sum(msg, dst, num_segments=N)
    cnt = jnp.maximum(
        jax.ops.segment_sum(jnp.ones((E,), _f32), dst, num_segments=N), 1.0)
    agg3 = agg / cnt[:, None]
    node3 = _node3_tc(t1, agg3, We2)

    csum = jax.ops.segment_sum(node3, batch, num_segments=BCR)
    ccnt = jnp.maximum(
        jax.ops.segment_sum(jnp.ones((N,), _f32), batch, num_segments=BCR),
        1.0)
    cf = csum / ccnt[:, None]
    cf = jnp.einsum('bij,bj->bi', feat_mask, cf)
    h = jax.nn.softplus(cf @ Ew1 + Eb1)
    out = (h @ Ew2 + Eb2).reshape(BCR, L)

    eq0 = equality[:, 0]
    eq1 = equality[:, 1]

    def avg_row(row, eq0r):
        def body1(j, row):
            mm = eq0r[j]
            c = jnp.sum(mm.astype(row.dtype))
            mean = jnp.sum(row * mm) / jnp.maximum(c, 1.0)
            return jnp.where(mm, mean, row)
        return lax.fori_loop(0, L, body1, row)

    row1 = jax.vmap(avg_row)(out, eq0)

    eq1T = jnp.transpose(eq1.astype(_f32), (1, 2, 0)).reshape(L * L * BCR)
    res = _sc_pairs(row1.T.reshape(L * BCR), eq1T)
    return jnp.transpose(res.reshape(L, BCR)).reshape(BCR, 6, 6)
